# speedup vs baseline: 7.7466x; 7.7466x over previous
import functools

import jax
import jax.numpy as jnp
from jax import lax
from jax.experimental import pallas as pl
from jax.experimental.pallas import tpu as pltpu
from jax.experimental.pallas import tpu_sc as plsc

VOCAB = 100000
D = 128
HD = 128
B = 16384
H = 20

NC = 2
NS = 16
NW = NC * NS
L = 16

B_PER_W = B // NW
CHUNK = 4
IDX_PER_CHUNK = CHUNK * H
NCHUNK = B_PER_W // CHUNK

_mesh = plsc.VectorSubcoreMesh(core_axis_name="c", subcore_axis_name="s")


@functools.partial(
    pl.kernel,
    out_type=jax.ShapeDtypeStruct((B, D), jnp.float32),
    mesh=_mesh,
    scratch_types=[
        pltpu.VMEM((NCHUNK, IDX_PER_CHUNK), jnp.int32),
        pltpu.VMEM((IDX_PER_CHUNK, D), jnp.float32),
        pltpu.VMEM((IDX_PER_CHUNK, D), jnp.float32),
        pltpu.VMEM((B_PER_W, D), jnp.float32),
        pltpu.SemaphoreType.DMA,
        pltpu.SemaphoreType.DMA,
    ],
)
def _sc_pool(x_hbm, table_hbm, out_hbm, idx_v, rows0, rows1, pooled_v, sem0, sem1):
    wid = lax.axis_index("s") * NC + lax.axis_index("c")

    pltpu.sync_copy(x_hbm.at[wid], idx_v)

    def start(chunk, rows_buf, sem):
        pltpu.async_copy(table_hbm.at[idx_v.at[chunk]], rows_buf, sem)

    def wait(chunk, rows_buf, sem):
        pltpu.make_async_copy(table_hbm.at[idx_v.at[chunk]], rows_buf, sem).wait()

    def reduce_chunk(rows_buf, chunk):
        @pl.loop(0, CHUNK)
        def _(c):
            @pl.loop(0, D, step=L)
            def _(d):
                acc = rows_buf[c * H, pl.ds(d, L)]
                for h in range(1, H):
                    acc = acc + rows_buf[c * H + h, pl.ds(d, L)]
                pooled_v[chunk * CHUNK + c, pl.ds(d, L)] = acc

    start(0, rows0, sem0)

    @pl.loop(0, NCHUNK, step=2)
    def _(i):
        start(i + 1, rows1, sem1)
        wait(i, rows0, sem0)
        reduce_chunk(rows0, i)

        @pl.when(i + 2 < NCHUNK)
        def _():
            start(i + 2, rows0, sem0)

        wait(i + 1, rows1, sem1)
        reduce_chunk(rows1, i + 1)

    pltpu.sync_copy(pooled_v, out_hbm.at[pl.ds(wid * B_PER_W, B_PER_W)])


def _head_body(pooled_ref, w1_ref, b1_ref, w2_ref, b2_ref, out_ref):
    p = pooled_ref[...]
    w1 = w1_ref[...] * (1.0 / H)
    h = jnp.dot(p, w1, preferred_element_type=jnp.float32) + b1_ref[...]
    h = jnp.maximum(h, 0.0)
    out = jnp.dot(h, w2_ref[...], preferred_element_type=jnp.float32)
    out_ref[...] = out + b2_ref[...]


_head = pl.pallas_call(
    _head_body,
    out_shape=jax.ShapeDtypeStruct((B, 1), jnp.float32),
)


def kernel(x, embed_table, W1, b1, W2, b2):
    x_tiles = x.reshape(NW, NCHUNK, IDX_PER_CHUNK)
    pooled = _sc_pool(x_tiles, embed_table)
    out = _head(pooled, W1, b1, W2, b2)
    return out[:, 0]


# speedup vs baseline: 8.6514x; 1.1168x over previous
import functools

import jax
import jax.numpy as jnp
from jax import lax
from jax.experimental import pallas as pl
from jax.experimental.pallas import tpu as pltpu
from jax.experimental.pallas import tpu_sc as plsc

VOCAB = 100000
D = 128
HD = 128
B = 16384
H = 20

NC = 2
NS = 16
NW = NC * NS
L = 16

B_PER_W = B // NW
CHUNK = 4
IDX_PER_CHUNK = CHUNK * H
NCHUNK = B_PER_W // CHUNK

_mesh = plsc.VectorSubcoreMesh(core_axis_name="c", subcore_axis_name="s")


@functools.partial(
    pl.kernel,
    out_type=jax.ShapeDtypeStruct((B, D), jnp.float32),
    mesh=_mesh,
    scratch_types=[
        pltpu.VMEM((NCHUNK, IDX_PER_CHUNK), jnp.int32),
        pltpu.VMEM((IDX_PER_CHUNK, D), jnp.float32),
        pltpu.VMEM((IDX_PER_CHUNK, D), jnp.float32),
        pltpu.VMEM((B_PER_W, D), jnp.float32),
        pltpu.SemaphoreType.DMA,
        pltpu.SemaphoreType.DMA,
    ],
)
def _sc_pool(x_hbm, table_hbm, out_hbm, idx_v, rows0, rows1, pooled_v, sem0, sem1):
    wid = lax.axis_index("s") * NC + lax.axis_index("c")

    pltpu.sync_copy(x_hbm.at[wid], idx_v)

    def start(chunk, rows_buf, sem):
        pltpu.async_copy(table_hbm.at[idx_v.at[chunk]], rows_buf, sem)

    def wait(chunk, rows_buf, sem):
        pltpu.make_async_copy(table_hbm.at[idx_v.at[chunk]], rows_buf, sem).wait()

    def reduce_chunk(rows_buf, chunk):
        @pl.loop(0, D, step=L)
        def _(d):
            for c in range(CHUNK):
                v = [rows_buf[c * H + h, pl.ds(d, L)] for h in range(H)]
                while len(v) > 1:
                    nxt = [v[i] + v[i + 1] for i in range(0, len(v) - 1, 2)]
                    if len(v) % 2:
                        nxt.append(v[-1])
                    v = nxt
                pooled_v[chunk * CHUNK + c, pl.ds(d, L)] = v[0]

    start(0, rows0, sem0)

    @pl.loop(0, NCHUNK, step=2)
    def _(i):
        start(i + 1, rows1, sem1)
        wait(i, rows0, sem0)
        reduce_chunk(rows0, i)

        @pl.when(i + 2 < NCHUNK)
        def _():
            start(i + 2, rows0, sem0)

        wait(i + 1, rows1, sem1)
        reduce_chunk(rows1, i + 1)

    pltpu.sync_copy(pooled_v, out_hbm.at[pl.ds(wid * B_PER_W, B_PER_W)])


def _head_body(pooled_ref, w1_ref, b1_ref, w2_ref, b2_ref, out_ref):
    p = pooled_ref[...]
    w1 = w1_ref[...] * (1.0 / H)
    h = jnp.dot(p, w1, preferred_element_type=jnp.float32) + b1_ref[...]
    h = jnp.maximum(h, 0.0)
    out = jnp.dot(h, w2_ref[...], preferred_element_type=jnp.float32)
    out_ref[...] = out + b2_ref[...]


_head = pl.pallas_call(
    _head_body,
    out_shape=jax.ShapeDtypeStruct((B, 1), jnp.float32),
)


def kernel(x, embed_table, W1, b1, W2, b2):
    x_tiles = x.reshape(NW, NCHUNK, IDX_PER_CHUNK)
    pooled = _sc_pool(x_tiles, embed_table)
    out = _head(pooled, W1, b1, W2, b2)
    return out[:, 0]


# speedup vs baseline: 11.0386x; 1.2759x over previous
import functools

import jax
import jax.numpy as jnp
from jax import lax
from jax.experimental import pallas as pl
from jax.experimental.pallas import tpu as pltpu
from jax.experimental.pallas import tpu_sc as plsc

VOCAB = 100000
D = 128
HD = 128
B = 16384
H = 20

NC = 2
NS = 16
NW = NC * NS
L = 16

B_PER_W = B // NW
CHUNK = 4
IDX_PER_CHUNK = CHUNK * H
NCHUNK = B_PER_W // CHUNK

_mesh = plsc.VectorSubcoreMesh(core_axis_name="c", subcore_axis_name="s")


@functools.partial(
    pl.kernel,
    out_type=jax.ShapeDtypeStruct((B, D), jnp.float32),
    mesh=_mesh,
    scratch_types=[
        pltpu.VMEM((NCHUNK, IDX_PER_CHUNK), jnp.int32),
        pltpu.VMEM((IDX_PER_CHUNK, D), jnp.float32),
        pltpu.VMEM((IDX_PER_CHUNK, D), jnp.float32),
        pltpu.VMEM((IDX_PER_CHUNK, D), jnp.float32),
        pltpu.VMEM((IDX_PER_CHUNK, D), jnp.float32),
        pltpu.VMEM((B_PER_W, D), jnp.float32),
        pltpu.SemaphoreType.DMA,
        pltpu.SemaphoreType.DMA,
        pltpu.SemaphoreType.DMA,
        pltpu.SemaphoreType.DMA,
    ],
)
def _sc_pool(x_hbm, table_hbm, out_hbm, idx_v, rows0, rows1, rows2, rows3,
             pooled_v, sem0, sem1, sem2, sem3):
    wid = lax.axis_index("s") * NC + lax.axis_index("c")

    pltpu.sync_copy(x_hbm.at[wid], idx_v)

    def start(chunk, rows_buf, sem):
        pltpu.async_copy(table_hbm.at[idx_v.at[chunk]], rows_buf, sem)

    def wait(chunk, rows_buf, sem):
        pltpu.make_async_copy(table_hbm.at[idx_v.at[chunk]], rows_buf, sem).wait()

    def reduce_chunk(rows_buf, chunk):
        @pl.loop(0, D, step=L)
        def _(d):
            for c in range(CHUNK):
                v = [rows_buf[c * H + h, pl.ds(d, L)] for h in range(H)]
                while len(v) > 1:
                    nxt = [v[i] + v[i + 1] for i in range(0, len(v) - 1, 2)]
                    if len(v) % 2:
                        nxt.append(v[-1])
                    v = nxt
                pooled_v[chunk * CHUNK + c, pl.ds(d, L)] = v[0]

    bufs = ((rows0, sem0), (rows1, sem1), (rows2, sem2), (rows3, sem3))
    NBUF = len(bufs)

    for b, (rows_b, sem_b) in enumerate(bufs):
        start(b, rows_b, sem_b)

    @pl.loop(0, NCHUNK, step=NBUF)
    def _(i):
        for b, (rows_b, sem_b) in enumerate(bufs):
            wait(i + b, rows_b, sem_b)
            reduce_chunk(rows_b, i + b)

            @pl.when(i + b + NBUF < NCHUNK)
            def _():
                start(i + b + NBUF, rows_b, sem_b)

    pltpu.sync_copy(pooled_v, out_hbm.at[pl.ds(wid * B_PER_W, B_PER_W)])


def _head_body(pooled_ref, w1_ref, b1_ref, w2_ref, b2_ref, out_ref):
    p = pooled_ref[...]
    w1 = w1_ref[...] * (1.0 / H)
    h = jnp.dot(p, w1, preferred_element_type=jnp.float32) + b1_ref[...]
    h = jnp.maximum(h, 0.0)
    out = jnp.dot(h, w2_ref[...], preferred_element_type=jnp.float32)
    out_ref[...] = out + b2_ref[...]


_head = pl.pallas_call(
    _head_body,
    out_shape=jax.ShapeDtypeStruct((B, 1), jnp.float32),
)


def kernel(x, embed_table, W1, b1, W2, b2):
    x_tiles = x.reshape(NW, NCHUNK, IDX_PER_CHUNK)
    pooled = _sc_pool(x_tiles, embed_table)
    out = _head(pooled, W1, b1, W2, b2)
    return out[:, 0]


# speedup vs baseline: 12.6526x; 1.1462x over previous
import functools

import jax
import jax.numpy as jnp
from jax import lax
from jax.experimental import pallas as pl
from jax.experimental.pallas import tpu as pltpu
from jax.experimental.pallas import tpu_sc as plsc

VOCAB = 100000
D = 128
HD = 128
B = 16384
H = 20

NC = 2
NS = 16
NW = NC * NS
L = 16

B_PER_W = B // NW
CHUNK = 4
IDX_PER_CHUNK = CHUNK * H
NCHUNK = B_PER_W // CHUNK
NBUF = 8
GROUP_ROWS = NBUF * CHUNK

_mesh = plsc.VectorSubcoreMesh(core_axis_name="c", subcore_axis_name="s")

_scratch = (
    [pltpu.VMEM((NCHUNK, IDX_PER_CHUNK), jnp.int32)]
    + [pltpu.VMEM((IDX_PER_CHUNK, D), jnp.float32)] * NBUF
    + [pltpu.VMEM((GROUP_ROWS, D), jnp.float32)] * 2
    + [pltpu.SemaphoreType.DMA] * NBUF
    + [pltpu.SemaphoreType.DMA] * 2
)


@functools.partial(
    pl.kernel,
    out_type=jax.ShapeDtypeStruct((B, D), jnp.float32),
    mesh=_mesh,
    scratch_types=_scratch,
)
def _sc_pool(x_hbm, table_hbm, out_hbm, idx_v, *refs):
    rows = refs[:NBUF]
    pooled = refs[NBUF:NBUF + 2]
    gsems = refs[NBUF + 2:2 * NBUF + 2]
    osems = refs[2 * NBUF + 2:]

    wid = lax.axis_index("s") * NC + lax.axis_index("c")
    out_base = wid * B_PER_W

    pltpu.sync_copy(x_hbm.at[wid], idx_v)

    def start(chunk, b):
        pltpu.async_copy(table_hbm.at[idx_v.at[chunk]], rows[b], gsems[b])

    def wait(chunk, b):
        pltpu.make_async_copy(table_hbm.at[idx_v.at[chunk]], rows[b], gsems[b]).wait()

    def out_slice(chunk):
        return out_hbm.at[pl.ds(out_base + chunk * CHUNK, GROUP_ROWS)]

    def reduce_chunk(rows_buf, pooled_buf, row_base):
        @pl.loop(0, D, step=L)
        def _(d):
            for c in range(CHUNK):
                v = [rows_buf[c * H + h, pl.ds(d, L)] for h in range(H)]
                while len(v) > 1:
                    nxt = [v[i] + v[i + 1] for i in range(0, len(v) - 1, 2)]
                    if len(v) % 2:
                        nxt.append(v[-1])
                    v = nxt
                pooled_buf[row_base + c, pl.ds(d, L)] = v[0]

    for b in range(NBUF):
        start(b, b)

    @pl.loop(0, NCHUNK, step=2 * NBUF)
    def _(i):
        for half in range(2):
            pooled_b, osem = pooled[half], osems[half]

            @pl.when(i > 0)
            def _():
                pltpu.make_async_copy(pooled_b, out_slice(i + half * NBUF), osem).wait()

            for b in range(NBUF):
                chunk = i + half * NBUF + b
                wait(chunk, b)
                reduce_chunk(rows[b], pooled_b, b * CHUNK)

                nxt = chunk + NBUF

                @pl.when(nxt < NCHUNK)
                def _():
                    start(nxt, b)

            pltpu.async_copy(pooled_b, out_slice(i + half * NBUF), osem)

    for half in range(2):
        pltpu.make_async_copy(pooled[half], out_slice(0), osems[half]).wait()


def _head_body(pooled_ref, w1_ref, b1_ref, w2_ref, b2_ref, out_ref):
    p = pooled_ref[...]
    w1 = w1_ref[...] * (1.0 / H)
    h = jnp.dot(p, w1, preferred_element_type=jnp.float32) + b1_ref[...]
    h = jnp.maximum(h, 0.0)
    out = jnp.dot(h, w2_ref[...], preferred_element_type=jnp.float32)
    out_ref[...] = out + b2_ref[...]


_head = pl.pallas_call(
    _head_body,
    out_shape=jax.ShapeDtypeStruct((B, 1), jnp.float32),
)


def kernel(x, embed_table, W1, b1, W2, b2):
    x_tiles = x.reshape(NW, NCHUNK, IDX_PER_CHUNK)
    pooled = _sc_pool(x_tiles, embed_table)
    out = _head(pooled, W1, b1, W2, b2)
    return out[:, 0]
